# baseline (speedup 1.0000x reference)
import functools

import jax
import jax.numpy as jnp
from jax import lax
from jax.experimental import pallas as pl
from jax.experimental.pallas import tpu as pltpu
from jax.experimental.pallas import tpu_sc as plsc

D_MODEL = 64
SCALE = 8.0

NUM_CORES = 2
NUM_SUBCORES = 16
NUM_WORKERS = NUM_CORES * NUM_SUBCORES
LANES = 16

CHUNK = 128
CHUNKS_PER_GROUP = 2
GROUP = CHUNK * CHUNKS_PER_GROUP
NBUF = 4
LOOKAHEAD = NBUF - 1


def _emb_body(n_groups, chunks_per_worker, x_hbm, table_hbm, out_hbm,
              idx_v, rows_v, *sems):
    gsems = sems[:NBUF]
    wsems = sems[NBUF:]

    wid = lax.axis_index("s") * NUM_CORES + lax.axis_index("c")
    chunk_row0 = wid * chunks_per_worker
    row0 = wid * (chunks_per_worker * CHUNK)

    pltpu.sync_copy(x_hbm.at[pl.ds(chunk_row0, chunks_per_worker)], idx_v)

    def fire_gather(g, b):
        for j in range(CHUNKS_PER_GROUP):
            pltpu.async_copy(
                table_hbm.at[idx_v.at[g * CHUNKS_PER_GROUP + j]],
                rows_v.at[b, pl.ds(j * CHUNK, CHUNK)],
                gsems[b],
            )

    def wait_gather(b):
        pltpu.make_async_copy(
            out_hbm.at[pl.ds(0, GROUP)], rows_v.at[b], gsems[b]
        ).wait()

    def fire_writeback(g, b):
        pltpu.async_copy(rows_v.at[b], out_hbm.at[pl.ds(row0 + g * GROUP, GROUP)],
                         wsems[b])

    def wait_writeback(b):
        pltpu.make_async_copy(
            rows_v.at[b], out_hbm.at[pl.ds(0, GROUP)], wsems[b]
        ).wait()

    def scale_rows(b):
        @plsc.parallel_loop(0, GROUP, step=4, unroll=2)
        def _scale(r):
            for rr in range(4):
                for c in range(D_MODEL // LANES):
                    sl = pl.ds(c * LANES, LANES)
                    rows_v[b, r + rr, sl] = rows_v[b, r + rr, sl] * SCALE

    for g in range(LOOKAHEAD):
        fire_gather(g, g % NBUF)

    @pl.loop(0, n_groups, step=NBUF)
    def _ring(t):
        for b0 in range(NBUF):
            g = t + b0
            gf = g + LOOKAHEAD
            bf = (b0 + LOOKAHEAD) % NBUF

            @pl.when(gf < n_groups)
            def _():
                @pl.when(g >= 1)
                def _():
                    wait_writeback(bf)
                fire_gather(gf, bf)

            wait_gather(b0)
            scale_rows(b0)
            fire_writeback(g, b0)

    for b in range(NBUF):
        wait_writeback(b)


def kernel(x, table):
    orig_shape = x.shape
    n = x.size
    assert n % (NUM_WORKERS * GROUP * NBUF) == 0
    chunks_per_worker = n // (NUM_WORKERS * CHUNK)
    n_groups = chunks_per_worker // CHUNKS_PER_GROUP

    x2d = x.reshape(n // CHUNK, CHUNK).astype(jnp.int32)

    mesh = plsc.VectorSubcoreMesh(core_axis_name="c", subcore_axis_name="s")
    out = pl.kernel(
        functools.partial(_emb_body, n_groups, chunks_per_worker),
        out_type=jax.ShapeDtypeStruct((n, D_MODEL), jnp.float32),
        mesh=mesh,
        compiler_params=pltpu.CompilerParams(
            use_tc_tiling_on_sc=False, skip_device_barrier=True
        ),
        scratch_types=[
            pltpu.VMEM((chunks_per_worker, CHUNK), jnp.int32),
            pltpu.VMEM((NBUF, GROUP, D_MODEL), jnp.float32),
        ] + [pltpu.SemaphoreType.DMA] * (2 * NBUF),
    )(x2d, table)
    return out.reshape(*orig_shape, D_MODEL)


# speedup vs baseline: 1.0249x; 1.0249x over previous
import functools

import jax
import jax.numpy as jnp
from jax import lax
from jax.experimental import pallas as pl
from jax.experimental.pallas import tpu as pltpu
from jax.experimental.pallas import tpu_sc as plsc

D_MODEL = 64
SCALE = 8.0

NUM_CORES = 2
NUM_SUBCORES = 16
NUM_WORKERS = NUM_CORES * NUM_SUBCORES
LANES = 16

CHUNK = 128
CHUNKS_PER_GROUP = 2
GROUP = CHUNK * CHUNKS_PER_GROUP
NBUF = 4
LOOKAHEAD = NBUF - 1


def _emb_body(n_groups, chunks_per_worker, x_hbm, table_hbm, out_hbm,
              idx_v, rows_v, *sems):
    gsems = sems[:NBUF]
    wsems = sems[NBUF:]

    wid = lax.axis_index("s") * NUM_CORES + lax.axis_index("c")
    chunk_row0 = wid * chunks_per_worker
    row0 = wid * (chunks_per_worker * CHUNK)

    pltpu.sync_copy(x_hbm.at[pl.ds(chunk_row0, chunks_per_worker)], idx_v)

    def fire_gather(g, b):
        for j in range(CHUNKS_PER_GROUP):
            pltpu.async_copy(
                table_hbm.at[idx_v.at[g * CHUNKS_PER_GROUP + j]],
                rows_v.at[b, pl.ds(j * CHUNK, CHUNK)],
                gsems[b],
            )

    def wait_gather(b):
        pltpu.make_async_copy(
            out_hbm.at[pl.ds(0, GROUP)], rows_v.at[b], gsems[b]
        ).wait()

    def fire_writeback(g, b):
        pltpu.async_copy(rows_v.at[b], out_hbm.at[pl.ds(row0 + g * GROUP, GROUP)],
                         wsems[b])

    def wait_writeback(b):
        pltpu.make_async_copy(
            rows_v.at[b], out_hbm.at[pl.ds(0, GROUP)], wsems[b]
        ).wait()

    def scale_rows(b):
        @plsc.parallel_loop(0, GROUP, step=4, unroll=2)
        def _scale(r):
            for rr in range(4):
                for c in range(D_MODEL // LANES):
                    sl = pl.ds(c * LANES, LANES)
                    rows_v[b, r + rr, sl] = rows_v[b, r + rr, sl] * SCALE

    for g in range(LOOKAHEAD):
        fire_gather(g, g % NBUF)

    @pl.loop(0, n_groups, step=NBUF)
    def _ring(t):
        for b0 in range(NBUF):
            g = t + b0
            gf = g + LOOKAHEAD
            bf = (b0 + LOOKAHEAD) % NBUF

            @pl.when(gf < n_groups)
            def _():
                @pl.when(g >= 1)
                def _():
                    wait_writeback(bf)
                fire_gather(gf, bf)

            wait_gather(b0)
            scale_rows(b0)
            fire_writeback(g, b0)

    for b in range(NBUF):
        wait_writeback(b)


def kernel(x, table):
    rows, cols = x.shape
    n = x.size
    assert n % (NUM_WORKERS * GROUP * NBUF) == 0
    chunks_per_worker = n // (NUM_WORKERS * CHUNK)
    n_groups = chunks_per_worker // CHUNKS_PER_GROUP

    x2d = x.T.reshape(n // CHUNK, CHUNK).astype(jnp.int32)

    mesh = plsc.VectorSubcoreMesh(core_axis_name="c", subcore_axis_name="s")
    out = pl.kernel(
        functools.partial(_emb_body, n_groups, chunks_per_worker),
        out_type=jax.ShapeDtypeStruct((n, D_MODEL), jnp.float32),
        mesh=mesh,
        compiler_params=pltpu.CompilerParams(
            use_tc_tiling_on_sc=False, skip_device_barrier=True
        ),
        scratch_types=[
            pltpu.VMEM((chunks_per_worker, CHUNK), jnp.int32),
            pltpu.VMEM((NBUF, GROUP, D_MODEL), jnp.float32),
        ] + [pltpu.SemaphoreType.DMA] * (2 * NBUF),
    )(x2d, table)
    return out.reshape(cols, rows, D_MODEL).transpose(1, 0, 2)


# speedup vs baseline: 1.2622x; 1.2315x over previous
import functools

import jax
import jax.numpy as jnp
from jax import lax
from jax.experimental import pallas as pl
from jax.experimental.pallas import tpu as pltpu
from jax.experimental.pallas import tpu_sc as plsc

D_MODEL = 64
D_PAD = 128
SCALE = 8.0

NUM_CORES = 2
NUM_SUBCORES = 16
NUM_WORKERS = NUM_CORES * NUM_SUBCORES
LANES = 16

CHUNK = 128
CHUNKS_PER_GROUP = 1
GROUP = CHUNK * CHUNKS_PER_GROUP
NBUF = 4
LOOKAHEAD = NBUF - 1


def _emb_body(n_groups, chunks_per_worker, x_hbm, table_hbm, out_hbm,
              idx_v, rows_v, *sems):
    gsems = sems[:NBUF]
    wsems = sems[NBUF:]

    wid = lax.axis_index("s") * NUM_CORES + lax.axis_index("c")
    chunk_row0 = wid * chunks_per_worker
    row0 = wid * (chunks_per_worker * CHUNK)

    pltpu.sync_copy(x_hbm.at[pl.ds(chunk_row0, chunks_per_worker)], idx_v)

    def fire_gather(g, b):
        for j in range(CHUNKS_PER_GROUP):
            pltpu.async_copy(
                table_hbm.at[idx_v.at[g * CHUNKS_PER_GROUP + j]],
                rows_v.at[b, pl.ds(j * CHUNK, CHUNK)],
                gsems[b],
            )

    def wait_gather(b):
        pltpu.make_async_copy(
            table_hbm.at[pl.ds(0, GROUP)], rows_v.at[b], gsems[b]
        ).wait()

    def fire_writeback(g, b):
        pltpu.async_copy(rows_v.at[b],
                         out_hbm.at[pl.ds(row0 + g * GROUP, GROUP)],
                         wsems[b])

    def wait_writeback(b):
        pltpu.make_async_copy(
            rows_v.at[b], out_hbm.at[pl.ds(0, GROUP)], wsems[b]
        ).wait()

    def scale_rows(b):
        @plsc.parallel_loop(0, GROUP, step=4, unroll=2)
        def _scale(r):
            for rr in range(4):
                for c in range(D_MODEL // LANES):
                    sl = pl.ds(c * LANES, LANES)
                    rows_v[b, r + rr, sl] = rows_v[b, r + rr, sl] * SCALE

    for g in range(LOOKAHEAD):
        fire_gather(g, g % NBUF)

    @pl.loop(0, n_groups, step=NBUF)
    def _ring(t):
        for b0 in range(NBUF):
            g = t + b0
            gf = g + LOOKAHEAD
            bf = (b0 + LOOKAHEAD) % NBUF

            @pl.when(gf < n_groups)
            def _():
                @pl.when(g >= 1)
                def _():
                    wait_writeback(bf)
                fire_gather(gf, bf)

            wait_gather(b0)
            scale_rows(b0)
            fire_writeback(g, b0)

    for b in range(NBUF):
        wait_writeback(b)


def kernel(x, table):
    rows, cols = x.shape
    n = x.size
    assert n % (NUM_WORKERS * GROUP * NBUF) == 0
    chunks_per_worker = n // (NUM_WORKERS * CHUNK)
    n_groups = chunks_per_worker // CHUNKS_PER_GROUP

    x2d = x.T.reshape(n // CHUNK, CHUNK).astype(jnp.int32)

    table_p = jnp.pad(table, ((0, 0), (0, D_PAD - D_MODEL)))

    mesh = plsc.VectorSubcoreMesh(core_axis_name="c", subcore_axis_name="s")
    out = pl.kernel(
        functools.partial(_emb_body, n_groups, chunks_per_worker),
        out_type=jax.ShapeDtypeStruct((n, D_PAD), jnp.float32),
        mesh=mesh,
        compiler_params=pltpu.CompilerParams(use_tc_tiling_on_sc=True),
        scratch_types=[
            pltpu.VMEM((chunks_per_worker, CHUNK), jnp.int32),
            pltpu.VMEM((NBUF, GROUP, D_PAD), jnp.float32),
        ] + [pltpu.SemaphoreType.DMA] * (2 * NBUF),
    )(x2d, table_p)
    out = out.reshape(cols, rows, D_PAD)[:, :, :D_MODEL]
    return out.transpose(1, 0, 2)
